# baseline (speedup 1.0000x reference)
import functools

import jax
import jax.numpy as jnp
from jax import lax
from jax.experimental import pallas as pl
from jax.experimental.pallas import tpu as pltpu
from jax.experimental.pallas import tpu_sc as plsc

_B, _C, _H, _W = 32, 384, 64, 64
_HW = _H * _W
_NC, _NS = 2, 16
_NW = _NC * _NS
_PERW = (_B * _C * _HW) // _NW
_CHUNK_CH = 2
_CHUNK = _CHUNK_CH * _HW
_NCHUNK = _C // _CHUNK_CH


@functools.partial(
    pl.kernel,
    out_type=jax.ShapeDtypeStruct((_B * _C * _HW,), jnp.float32),
    mesh=plsc.VectorSubcoreMesh(
        core_axis_name="c", subcore_axis_name="s",
        num_cores=_NC, num_subcores=_NS,
    ),
    scratch_types=[
        pltpu.VMEM((8 * _C + 16,), jnp.float32),
        pltpu.VMEM((16,), jnp.int32),
        pltpu.VMEM((_CHUNK,), jnp.float32),
        pltpu.VMEM((_CHUNK,), jnp.float32),
        pltpu.VMEM((_CHUNK,), jnp.float32),
        pltpu.VMEM((_CHUNK,), jnp.float32),
        pltpu.SemaphoreType.DMA,
        pltpu.SemaphoreType.DMA,
        pltpu.SemaphoreType.DMA,
        pltpu.SemaphoreType.DMA,
    ],
)
def _sc_scale(x_hbm, st_hbm, e_hbm, o_hbm,
              emb_v, st_s, in0, in1, out0, out1, si0, si1, so0, so1):
    wid = lax.axis_index("s") * _NC + lax.axis_index("c")
    base = wid * _PERW
    pltpu.sync_copy(st_hbm, st_s)
    pltpu.sync_copy(e_hbm, emb_v.at[pl.ds(0, 8 * _C)])
    st = st_s[...][0]

    ins = (in0, in1)
    outs = (out0, out1)
    isems = (si0, si1)
    osems = (so0, so1)

    def start_in(k, b):
        pltpu.async_copy(x_hbm.at[pl.ds(base + k * _CHUNK, _CHUNK)],
                         ins[b], isems[b])

    def wait_in(b):
        pltpu.make_async_copy(x_hbm.at[pl.ds(base, _CHUNK)],
                              ins[b], isems[b]).wait()

    def start_out(k, b):
        pltpu.async_copy(outs[b],
                         o_hbm.at[pl.ds(base + k * _CHUNK, _CHUNK)], osems[b])

    def wait_out(b):
        pltpu.make_async_copy(outs[b],
                              o_hbm.at[pl.ds(base, _CHUNK)], osems[b]).wait()

    def compute(k, b):
        inb, outb = ins[b], outs[b]
        for ch in range(_CHUNK_CH):
            scv = emb_v[pl.ds(st * _C + k * _CHUNK_CH + ch, 16)]
            sc = scv[0]
            for v in range(_HW // 16):
                sl = pl.ds(ch * _HW + v * 16, 16)
                outb[sl] = inb[sl] * sc

    start_in(0, 0)

    def pair(k2, carry):
        k = k2 * 2

        wait_in(0)

        @pl.when(k + 1 < _NCHUNK)
        def _():
            start_in(k + 1, 1)

        @pl.when(k2 > 0)
        def _():
            wait_out(0)

        compute(k, 0)
        start_out(k, 0)

        wait_in(1)

        @pl.when(k + 2 < _NCHUNK)
        def _():
            start_in(k + 2, 0)

        @pl.when(k2 > 0)
        def _():
            wait_out(1)

        compute(k + 1, 1)
        start_out(k + 1, 1)
        return carry

    lax.fori_loop(0, _NCHUNK // 2, pair, 0)
    wait_out(0)
    wait_out(1)


def kernel(x, stage, embeddings):
    s = jnp.full((16,), stage, dtype=jnp.int32)
    out = _sc_scale(x.reshape(-1), s, embeddings.reshape(-1))
    return out.reshape(_B, _C, _H, _W)


# speedup vs baseline: 2.4319x; 2.4319x over previous
import functools

import jax
import jax.numpy as jnp
from jax import lax
from jax.experimental import pallas as pl
from jax.experimental.pallas import tpu as pltpu
from jax.experimental.pallas import tpu_sc as plsc

_B, _C, _H, _W = 32, 384, 64, 64
_HW = _H * _W
_NC, _NS = 2, 16


@functools.partial(
    pl.kernel,
    out_type=jax.ShapeDtypeStruct((_C,), jnp.float32),
    mesh=plsc.VectorSubcoreMesh(
        core_axis_name="c", subcore_axis_name="s",
        num_cores=_NC, num_subcores=_NS,
    ),
    scratch_types=[
        pltpu.VMEM((16,), jnp.int32),
        pltpu.VMEM((8 * _C,), jnp.float32),
    ],
)
def _sc_gather(st_hbm, e_hbm, o_hbm, st_s, emb_v):
    wid = lax.axis_index("s") * _NC + lax.axis_index("c")

    @pl.when(wid == 0)
    def _():
        pltpu.sync_copy(st_hbm, st_s)
        pltpu.sync_copy(e_hbm, emb_v)
        st = st_s[...][0]
        pltpu.sync_copy(emb_v.at[pl.ds(st * _C, _C)], o_hbm)


def _tc_body(x_ref, e_ref, o_ref):
    o_ref[...] = x_ref[...] * e_ref[...]


def _tc_scale(x3, scale3):
    return pl.pallas_call(
        _tc_body,
        grid=(_B,),
        in_specs=[
            pl.BlockSpec((1, _C, _HW), lambda i: (i, 0, 0)),
            pl.BlockSpec((1, _C, 1), lambda i: (0, 0, 0)),
        ],
        out_specs=pl.BlockSpec((1, _C, _HW), lambda i: (i, 0, 0)),
        out_shape=jax.ShapeDtypeStruct((_B, _C, _HW), jnp.float32),
        compiler_params=pltpu.CompilerParams(
            dimension_semantics=("arbitrary",),
        ),
    )(x3, scale3)


def kernel(x, stage, embeddings):
    s = jnp.full((16,), stage, dtype=jnp.int32)
    scale = _sc_gather(s, embeddings.reshape(-1))
    out = _tc_scale(x.reshape(_B, _C, _HW), scale.reshape(1, _C, 1))
    return out.reshape(_B, _C, _H, _W)
